# baseline (speedup 1.0000x reference)
import functools

import jax
import jax.numpy as jnp
from jax.experimental import pallas as pl

B, C_IN, C_OUT, H, W = 2, 192, 192, 32, 32
BREADTH, K = 8, 3
HW = H * W


def _fold_weights(weight):
    w = weight
    r0 = jnp.stack([w[..., 0, :], w[..., 1, :] + w[..., 2, :]], axis=3)
    r1 = jnp.stack([w[..., 0, :] + w[..., 1, :], w[..., 2, :]], axis=3)
    rw = jnp.stack([r0, r1], axis=3)
    c0 = jnp.stack([rw[..., 0], rw[..., 1] + rw[..., 2]], axis=-1)
    c1 = jnp.stack([rw[..., 0] + rw[..., 1], rw[..., 2]], axis=-1)
    cw = jnp.stack([c0, c1], axis=-2)
    wf = jnp.transpose(cw, (3, 5, 4, 6, 1, 2, 0))
    return wf.reshape(2, 2, 4 * C_IN, BREADTH * C_OUT)


def _phase_kernel(xpad_ref, cwt_ref, cb_ref, wf_ref, bias_ref, out_ref):
    xc = xpad_ref[0, 1 : H + 1, 1 : W + 1, :].reshape(HW, C_IN)
    logits = jnp.dot(xc, cwt_ref[...], preferred_element_type=jnp.float32)
    logits = logits + cb_ref[...]
    sel = jnp.argmax(logits, axis=-1)
    masks = [(sel == e).astype(jnp.float32)[:, None] for e in range(BREADTH)]

    for di in (0, 1):
        for dj in (0, 1):
            taps = []
            for r2 in (0, 1):
                for s2 in (0, 1):
                    t = xpad_ref[0, di + r2 : di + r2 + H, dj + s2 : dj + s2 + W, :]
                    taps.append(t.reshape(HW, C_IN))
            x4 = jnp.concatenate(taps, axis=-1)

            y = jnp.dot(x4, wf_ref[di, dj], preferred_element_type=jnp.float32)

            acc = bias_ref[...] * jnp.ones((HW, C_OUT), jnp.float32)
            for e in range(BREADTH):
                acc = acc + masks[e] * y[:, e * C_OUT : (e + 1) * C_OUT]
            out_ref[di, dj, 0] = acc.reshape(H, W, C_OUT)


@functools.partial(jax.jit, static_argnames=())
def kernel(x, coupler_w, coupler_b, weight, bias):
    xt = jnp.transpose(x, (0, 2, 3, 1))
    xpad = jnp.pad(xt, ((0, 0), (1, 1), (1, 1), (0, 0)))
    wf = _fold_weights(weight)
    cwt = coupler_w.T
    cb = coupler_b[None, :]
    b2 = bias[None, :]

    out6 = pl.pallas_call(
        _phase_kernel,
        grid=(B,),
        in_specs=[
            pl.BlockSpec((1, H + 2, W + 2, C_IN), lambda b: (b, 0, 0, 0)),
            pl.BlockSpec((C_IN, BREADTH), lambda b: (0, 0)),
            pl.BlockSpec((1, BREADTH), lambda b: (0, 0)),
            pl.BlockSpec((2, 2, 4 * C_IN, BREADTH * C_OUT),
                         lambda b: (0, 0, 0, 0)),
            pl.BlockSpec((1, C_OUT), lambda b: (0, 0)),
        ],
        out_specs=pl.BlockSpec((2, 2, 1, H, W, C_OUT),
                               lambda b: (0, 0, b, 0, 0, 0)),
        out_shape=jax.ShapeDtypeStruct((2, 2, B, H, W, C_OUT), jnp.float32),
    )(xpad, cwt, cb, wf, b2)

    out = jnp.transpose(out6, (2, 5, 3, 0, 4, 1))
    return out.reshape(B, C_OUT, 2 * H, 2 * W)


# speedup vs baseline: 2.2932x; 2.2932x over previous
import functools

import jax
import jax.numpy as jnp
from jax.experimental import pallas as pl

B, C_IN, C_OUT, H, W = 2, 192, 192, 32, 32
BREADTH, K = 8, 3
HW = H * W


def _fold_weights(weight):
    usets = {(0, 0): (0,), (0, 1): (1, 2), (1, 0): (0, 1), (1, 1): (2,)}
    parts = []
    for di in (0, 1):
        for dj in (0, 1):
            for r2 in (0, 1):
                for s2 in (0, 1):
                    acc = None
                    for u in usets[(di, r2)]:
                        for v in usets[(dj, s2)]:
                            t = weight[:, :, :, u, v]
                            acc = t if acc is None else acc + t
                    parts.append(acc)
    cw = jnp.stack(parts, axis=0)
    cw = cw.reshape(16, C_OUT, C_IN * BREADTH)
    wf = jnp.transpose(cw, (0, 2, 1))
    return wf.reshape(2, 2, 2, 2, C_IN, BREADTH, C_OUT).reshape(
        2, 2, 4 * C_IN, BREADTH * C_OUT)


def _phase_kernel(xpad_ref, xbf_ref, cwt_ref, cb_ref, wf_ref, bias_ref, out_ref):
    xc = xpad_ref[0, 1 : H + 1, 1 : W + 1, :].reshape(HW, C_IN)
    logits = jnp.dot(xc, cwt_ref[...], preferred_element_type=jnp.float32)
    logits = logits + cb_ref[...]
    sel = jnp.argmax(logits, axis=-1)
    masks = [(sel == e).astype(jnp.float32)[:, None] for e in range(BREADTH)]

    for di in (0, 1):
        for dj in (0, 1):
            taps = []
            for r2 in (0, 1):
                for s2 in (0, 1):
                    t = xbf_ref[0, di + r2 : di + r2 + H, dj + s2 : dj + s2 + W, :]
                    taps.append(t.reshape(HW, C_IN))
            x4 = jnp.concatenate(taps, axis=-1)

            y = jnp.dot(x4, wf_ref[di, dj], preferred_element_type=jnp.float32)

            acc = bias_ref[...] * jnp.ones((HW, C_OUT), jnp.float32)
            for e in range(BREADTH):
                acc = acc + masks[e] * y[:, e * C_OUT : (e + 1) * C_OUT]
            out_ref[di, dj, 0] = acc.reshape(H, W, C_OUT)


@functools.partial(jax.jit, static_argnames=())
def kernel(x, coupler_w, coupler_b, weight, bias):
    xt = jnp.transpose(x, (0, 2, 3, 1))
    xpad = jnp.pad(xt, ((0, 0), (1, 1), (1, 1), (0, 0)))
    xbf = xpad.astype(jnp.bfloat16)
    wf = _fold_weights(weight.astype(jnp.bfloat16))
    cwt = coupler_w.T
    cb = coupler_b[None, :]
    b2 = bias[None, :]

    out6 = pl.pallas_call(
        _phase_kernel,
        grid=(B,),
        in_specs=[
            pl.BlockSpec((1, H + 2, W + 2, C_IN), lambda b: (b, 0, 0, 0)),
            pl.BlockSpec((1, H + 2, W + 2, C_IN), lambda b: (b, 0, 0, 0)),
            pl.BlockSpec((C_IN, BREADTH), lambda b: (0, 0)),
            pl.BlockSpec((1, BREADTH), lambda b: (0, 0)),
            pl.BlockSpec((2, 2, 4 * C_IN, BREADTH * C_OUT),
                         lambda b: (0, 0, 0, 0)),
            pl.BlockSpec((1, C_OUT), lambda b: (0, 0)),
        ],
        out_specs=pl.BlockSpec((2, 2, 1, H, W, C_OUT),
                               lambda b: (0, 0, b, 0, 0, 0)),
        out_shape=jax.ShapeDtypeStruct((2, 2, B, H, W, C_OUT), jnp.float32),
    )(xpad, xbf, cwt, cb, wf, b2)

    out = jnp.transpose(out6, (2, 5, 3, 0, 4, 1))
    return out.reshape(B, C_OUT, 2 * H, 2 * W)
